# baseline (speedup 1.0000x reference)
import functools

import jax
import jax.numpy as jnp
from jax import lax
from jax.experimental import pallas as pl
from jax.experimental.pallas import tpu as pltpu
from jax.experimental.pallas import tpu_sc as plsc

EMB_DIM = 128
L = 200
LH = L // 2
NC, NS = 2, 16
NW = NC * NS
G = 32
NBUF = 4
NLG = (L + 15) // 16


def _sc_kernel(B):
    bpw = B // NW
    ng = bpw // G
    mesh = plsc.VectorSubcoreMesh(
        core_axis_name="c", subcore_axis_name="s", num_cores=NC,
        num_subcores=NS)

    @functools.partial(
        pl.kernel,
        out_type=jax.ShapeDtypeStruct((B, L), jnp.float32),
        mesh=mesh,
        compiler_params=pltpu.CompilerParams(needs_layout_passes=False),
        scratch_types=[
            pltpu.VMEM((2 * G, LH), jnp.int32),
            pltpu.VMEM((G,), jnp.int32),
            pltpu.VMEM((G, EMB_DIM), jnp.float32),
            [pltpu.VMEM((L, EMB_DIM), jnp.float32) for _ in range(NBUF)],
            pltpu.VMEM((G, L), jnp.float32),
            pltpu.SemaphoreType.DMA,
            [pltpu.SemaphoreType.DMA for _ in range(NBUF)],
        ],
    )
    def k(cen_hbm, ctx_hbm, v_hbm, u_hbm, out_hbm, ctx_v, cen_v, vrows,
          ubufs, obuf, sem_v, sems):
        wid = lax.axis_index("s") * NC + lax.axis_index("c")
        lanes = lax.iota(jnp.int32, 16)

        def start_u(p, b):
            pltpu.async_copy(u_hbm.at[ctx_v.at[2 * b]],
                             ubufs[p].at[pl.ds(0, LH)], sems[p])
            pltpu.async_copy(u_hbm.at[ctx_v.at[2 * b + 1]],
                             ubufs[p].at[pl.ds(LH, LH)], sems[p])

        def wait_u(p):
            pltpu.make_async_copy(u_hbm.at[ctx_v.at[0]],
                                  ubufs[p].at[pl.ds(0, LH)], sems[p]).wait()
            pltpu.make_async_copy(u_hbm.at[ctx_v.at[0]],
                                  ubufs[p].at[pl.ds(LH, LH)], sems[p]).wait()

        def compute(bb, buf):
            vc = [vrows[bb, pl.ds(c * 16, 16)] for c in range(8)]

            @plsc.parallel_loop(0, NLG, 1, unroll=2)
            def lg_body(lg):
                l0 = jnp.minimum(lg * 16, L - 16)
                r = []
                for j in range(16):
                    l = l0 + j
                    p = [vc[c] * buf[l, pl.ds(c * 16, 16)] for c in range(8)]
                    s0 = (p[0] + p[1]) + (p[2] + p[3])
                    s1 = (p[4] + p[5]) + (p[6] + p[7])
                    s = jnp.sum(s0 + s1)
                    r.append(jnp.where(lanes == j, s, 0.0))
                t0 = [r[2 * i] + r[2 * i + 1] for i in range(8)]
                t1 = [t0[2 * i] + t0[2 * i + 1] for i in range(4)]
                t2 = [t1[2 * i] + t1[2 * i + 1] for i in range(2)]
                obuf[bb, pl.ds(l0, 16)] = t2[0] + t2[1]

        def group_body(g, _):
            base = wid * bpw + g * G
            pltpu.sync_copy(cen_hbm.at[pl.ds(base, G)], cen_v)
            pltpu.sync_copy(ctx_hbm.at[pl.ds(2 * base, 2 * G)], ctx_v)
            pltpu.async_copy(v_hbm.at[cen_v], vrows, sem_v).wait()

            for p in range(NBUF):
                start_u(p, p)

            def ring_body(ii, _):
                bb = NBUF * ii
                for p in range(NBUF):
                    wait_u(p)
                    compute(bb + p, ubufs[p])
                    start_u(p, bb + NBUF + p)
                return 0

            lax.fori_loop(0, G // NBUF - 1, ring_body, 0)
            for p in range(NBUF):
                wait_u(p)
                compute(G - NBUF + p, ubufs[p])
            pltpu.sync_copy(obuf, out_hbm.at[pl.ds(base, G)])
            return 0

        lax.fori_loop(0, ng, group_body, 0)

    return k


def kernel(centers, contexts_negatives, V, U):
    B = centers.shape[0]
    cen = centers.reshape(B).astype(jnp.int32)
    ctx = contexts_negatives.astype(jnp.int32).reshape(2 * B, LH)
    out = _sc_kernel(B)(cen, ctx, V, U)
    return out.reshape(B, 1, L)


# speedup vs baseline: 1.3912x; 1.3912x over previous
import functools

import jax
import jax.numpy as jnp
from jax import lax
from jax.experimental import pallas as pl
from jax.experimental.pallas import tpu as pltpu
from jax.experimental.pallas import tpu_sc as plsc

EMB_DIM = 128
L = 200
LH = L // 2
NC, NS = 2, 16
NW = NC * NS
G = 64
NBUF = 2
NLG = (L + 15) // 16


def _sc_kernel(B):
    bpw = B // NW
    ng = bpw // G
    mesh = plsc.VectorSubcoreMesh(
        core_axis_name="c", subcore_axis_name="s", num_cores=NC,
        num_subcores=NS)

    @functools.partial(
        pl.kernel,
        out_type=jax.ShapeDtypeStruct((B, L), jnp.float32),
        mesh=mesh,
        compiler_params=pltpu.CompilerParams(needs_layout_passes=False),
        scratch_types=[
            pltpu.VMEM((2 * G, LH), jnp.int32),
            pltpu.VMEM((G,), jnp.int32),
            pltpu.VMEM((G, EMB_DIM), jnp.float32),
            [pltpu.VMEM((L, EMB_DIM), jnp.float32) for _ in range(NBUF)],
            pltpu.VMEM((G, L), jnp.float32),
            pltpu.SemaphoreType.DMA,
            [pltpu.SemaphoreType.DMA for _ in range(NBUF)],
        ],
    )
    def k(cen_hbm, ctx_hbm, v_hbm, u_hbm, out_hbm, ctx_v, cen_v, vrows,
          ubufs, obuf, sem_v, sems):
        wid = lax.axis_index("s") * NC + lax.axis_index("c")
        lanes = lax.iota(jnp.int32, 16)

        def start_u(p, b):
            pltpu.async_copy(u_hbm.at[ctx_v.at[2 * b]],
                             ubufs[p].at[pl.ds(0, LH)], sems[p])
            pltpu.async_copy(u_hbm.at[ctx_v.at[2 * b + 1]],
                             ubufs[p].at[pl.ds(LH, LH)], sems[p])

        def wait_u(p):
            pltpu.make_async_copy(u_hbm.at[ctx_v.at[0]],
                                  ubufs[p].at[pl.ds(0, LH)], sems[p]).wait()
            pltpu.make_async_copy(u_hbm.at[ctx_v.at[0]],
                                  ubufs[p].at[pl.ds(LH, LH)], sems[p]).wait()

        def compute(bb, buf):
            vc = [vrows[bb, pl.ds(c * 16, 16)] for c in range(8)]

            @plsc.parallel_loop(0, NLG, 1)
            def lg_body(lg):
                l0 = jnp.minimum(lg * 16, L - 16)
                r = []
                for j in range(16):
                    l = l0 + j
                    p = [vc[c] * buf[l, pl.ds(c * 16, 16)] for c in range(8)]
                    s0 = (p[0] + p[1]) + (p[2] + p[3])
                    s1 = (p[4] + p[5]) + (p[6] + p[7])
                    s = jnp.sum(s0 + s1)
                    r.append(jnp.where(lanes == j, s, 0.0))
                t0 = [r[2 * i] + r[2 * i + 1] for i in range(8)]
                t1 = [t0[2 * i] + t0[2 * i + 1] for i in range(4)]
                t2 = [t1[2 * i] + t1[2 * i + 1] for i in range(2)]
                obuf[bb, pl.ds(l0, 16)] = t2[0] + t2[1]

        def group_body(g, _):
            base = wid * bpw + g * G
            pltpu.sync_copy(cen_hbm.at[pl.ds(base, G)], cen_v)
            pltpu.sync_copy(ctx_hbm.at[pl.ds(2 * base, 2 * G)], ctx_v)
            pltpu.async_copy(v_hbm.at[cen_v], vrows, sem_v).wait()

            for p in range(NBUF):
                start_u(p, p)

            def ring_body(ii, _):
                bb = NBUF * ii
                for p in range(NBUF):
                    wait_u(p)
                    compute(bb + p, ubufs[p])
                    start_u(p, bb + NBUF + p)
                return 0

            lax.fori_loop(0, G // NBUF - 1, ring_body, 0)
            for p in range(NBUF):
                wait_u(p)
                compute(G - NBUF + p, ubufs[p])
            pltpu.sync_copy(obuf, out_hbm.at[pl.ds(base, G)])
            return 0

        lax.fori_loop(0, ng, group_body, 0)

    return k


def kernel(centers, contexts_negatives, V, U):
    B = centers.shape[0]
    cen = centers.reshape(B).astype(jnp.int32)
    ctx = contexts_negatives.astype(jnp.int32).reshape(2 * B, LH)
    out = _sc_kernel(B)(cen, ctx, V, U)
    return out.reshape(B, 1, L)


# speedup vs baseline: 1.5110x; 1.0861x over previous
import functools

import jax
import jax.numpy as jnp
from jax import lax
from jax.experimental import pallas as pl
from jax.experimental.pallas import tpu as pltpu
from jax.experimental.pallas import tpu_sc as plsc

EMB_DIM = 128
L = 200
LH = L // 2
NC, NS = 2, 16
NW = NC * NS
G = 16
NBUF = 4
NRING = G // NBUF
NLG = (L + 15) // 16


def _sc_kernel(B):
    bpw = B // NW
    ng = bpw // G
    mesh = plsc.VectorSubcoreMesh(
        core_axis_name="c", subcore_axis_name="s", num_cores=NC,
        num_subcores=NS)

    @functools.partial(
        pl.kernel,
        out_type=jax.ShapeDtypeStruct((B, L), jnp.float32),
        mesh=mesh,
        compiler_params=pltpu.CompilerParams(needs_layout_passes=False),
        scratch_types=[
            [pltpu.VMEM((2 * G, LH), jnp.int32) for _ in range(2)],
            [pltpu.VMEM((G,), jnp.int32) for _ in range(2)],
            [pltpu.VMEM((G, EMB_DIM), jnp.float32) for _ in range(2)],
            [pltpu.VMEM((G, L), jnp.float32) for _ in range(2)],
            [pltpu.VMEM((L, EMB_DIM), jnp.float32) for _ in range(NBUF)],
            pltpu.SemaphoreType.DMA,
            pltpu.SemaphoreType.DMA,
            [pltpu.SemaphoreType.DMA for _ in range(2)],
            [pltpu.SemaphoreType.DMA for _ in range(NBUF)],
        ],
    )
    def k(cen_hbm, ctx_hbm, v_hbm, u_hbm, out_hbm, ctxs, cens, vrows, obufs,
          ubufs, sem_st, sem_v, sem_outs, sems):
        wid = lax.axis_index("s") * NC + lax.axis_index("c")
        lanes = lax.iota(jnp.int32, 16)

        def start_u(ctx_v, p, b):
            pltpu.async_copy(u_hbm.at[ctx_v.at[2 * b]],
                             ubufs[p].at[pl.ds(0, LH)], sems[p])
            pltpu.async_copy(u_hbm.at[ctx_v.at[2 * b + 1]],
                             ubufs[p].at[pl.ds(LH, LH)], sems[p])

        def wait_u(p):
            pltpu.make_async_copy(u_hbm.at[ctxs[0].at[0]],
                                  ubufs[p].at[pl.ds(0, LH)], sems[p]).wait()
            pltpu.make_async_copy(u_hbm.at[ctxs[0].at[0]],
                                  ubufs[p].at[pl.ds(LH, LH)], sems[p]).wait()

        def stage_next(par2, nbase):
            pltpu.async_copy(ctx_hbm.at[pl.ds(2 * nbase, 2 * G)],
                             ctxs[par2], sem_st)
            pltpu.async_copy(cen_hbm.at[pl.ds(nbase, G)], cens[par2], sem_st)

        def wait_stage_start_vrows(par2):
            pltpu.make_async_copy(ctx_hbm.at[pl.ds(0, 2 * G)],
                                  ctxs[par2], sem_st).wait()
            pltpu.make_async_copy(cen_hbm.at[pl.ds(0, G)],
                                  cens[par2], sem_st).wait()
            pltpu.async_copy(v_hbm.at[cens[par2]], vrows[par2], sem_v)

        def wait_vrows(par2):
            pltpu.make_async_copy(v_hbm.at[pl.ds(0, G)],
                                  vrows[par2], sem_v).wait()

        def compute(par, bb, buf):
            vc = [vrows[par][bb, pl.ds(c * 16, 16)] for c in range(8)]

            @plsc.parallel_loop(0, NLG, 1)
            def lg_body(lg):
                l0 = jnp.minimum(lg * 16, L - 16)
                r = []
                for j in range(16):
                    l = l0 + j
                    p = [vc[c] * buf[l, pl.ds(c * 16, 16)] for c in range(8)]
                    s0 = (p[0] + p[1]) + (p[2] + p[3])
                    s1 = (p[4] + p[5]) + (p[6] + p[7])
                    s = jnp.sum(s0 + s1)
                    r.append(jnp.where(lanes == j, s, 0.0))
                t0 = [r[2 * i] + r[2 * i + 1] for i in range(8)]
                t1 = [t0[2 * i] + t0[2 * i + 1] for i in range(4)]
                t2 = [t1[2 * i] + t1[2 * i + 1] for i in range(2)]
                obufs[par][bb, pl.ds(l0, 16)] = t2[0] + t2[1]

        def phase(par, g):
            base = wid * bpw + g * G
            have_next = g + 1 < ng

            @pl.when(g >= 2)
            def _():
                pltpu.make_async_copy(obufs[par], out_hbm.at[pl.ds(0, G)],
                                      sem_outs[par]).wait()

            @pl.when(have_next)
            def _():
                stage_next(1 - par, base + G)

            def ring_body(ii, _):
                bb = NBUF * ii

                @pl.when((ii == NRING - 1) & have_next)
                def _():
                    wait_stage_start_vrows(1 - par)

                for p in range(NBUF):
                    wait_u(p)
                    compute(par, bb + p, ubufs[p])

                    @pl.when(ii < NRING - 1)
                    def _():
                        start_u(ctxs[par], p, bb + NBUF + p)

                    @pl.when((ii == NRING - 1) & have_next)
                    def _():
                        start_u(ctxs[1 - par], p, p)

                return 0

            lax.fori_loop(0, NRING, ring_body, 0)

            @pl.when(have_next)
            def _():
                wait_vrows(1 - par)

            pltpu.async_copy(obufs[par], out_hbm.at[pl.ds(base, G)],
                             sem_outs[par])

        pltpu.sync_copy(ctx_hbm.at[pl.ds(2 * wid * bpw, 2 * G)], ctxs[0])
        pltpu.sync_copy(cen_hbm.at[pl.ds(wid * bpw, G)], cens[0])
        pltpu.async_copy(v_hbm.at[cens[0]], vrows[0], sem_v)
        wait_vrows(0)
        for p in range(NBUF):
            start_u(ctxs[0], p, p)

        def pair_body(gg, _):
            phase(0, 2 * gg)
            phase(1, 2 * gg + 1)
            return 0

        lax.fori_loop(0, ng // 2, pair_body, 0)
        for par in range(2):
            pltpu.make_async_copy(obufs[par], out_hbm.at[pl.ds(0, G)],
                                  sem_outs[par]).wait()

    return k


def kernel(centers, contexts_negatives, V, U):
    B = centers.shape[0]
    cen = centers.reshape(B).astype(jnp.int32)
    ctx = contexts_negatives.astype(jnp.int32).reshape(2 * B, LH)
    out = _sc_kernel(B)(cen, ctx, V, U)
    return out.reshape(B, 1, L)
